# baseline (speedup 1.0000x reference)
import jax
import jax.numpy as jnp
from jax.experimental import pallas as pl
from jax.experimental.pallas import tpu as pltpu

_T = 12
_SEG = 256


def _fused_body(a_hbm, x_hbm, w_ref, bg_ref, cw_ref, cb_ref, o_ref,
                a_vm, x_vm, asem, xsem):
    n = a_hbm.shape[0]
    BT = x_hbm.shape[1]
    B = BT // _T

    acp = pltpu.make_async_copy(a_hbm, a_vm, asem)
    acp.start()
    xcp = pltpu.make_async_copy(x_hbm, x_vm, xsem)
    xcp.start()

    w = w_ref[...]
    rb = jax.lax.broadcasted_iota(jnp.int32, (BT, BT), 0) // _T
    cb = jax.lax.broadcasted_iota(jnp.int32, (BT, BT), 1) // _T
    blockmask = rb == cb
    zseg = jnp.zeros((BT, _SEG - BT), dtype=w.dtype)

    def bd(wk):
        tile = jnp.concatenate([jnp.concatenate([wk] * B, axis=1)] * B, axis=0)
        return jnp.concatenate(
            [jnp.where(blockmask, tile, 0.0), zseg], axis=1)

    WH = jnp.concatenate([bd(w[0] - w[2]), bd(w[1]), bd(w[2])], axis=1)

    xcp.wait()
    x = x_vm[...].astype(jnp.float32)
    ual = jnp.dot(x, WH, preferred_element_type=jnp.float32)
    q = ual[:, 0:BT]
    u1 = ual[:, _SEG:_SEG + BT]
    u2 = ual[:, 2 * _SEG:2 * _SEG + BT]

    acp.wait()
    A = a_vm[...]
    rown = jax.lax.broadcasted_iota(jnp.int32, (n, n), 0)
    coln = jax.lax.broadcasted_iota(jnp.int32, (n, n), 1)
    A0 = jnp.where(rown == coln, 0.0, A)
    deg = jnp.sum(A0, axis=1, keepdims=True)
    d = jnp.where(deg > 0, jax.lax.rsqrt(deg), 0.0)

    v = -d * jnp.dot(A0, d * u2, preferred_element_type=jnp.float32)
    p = u1 + 2.0 * v
    w2 = -d * jnp.dot(A0, d * p, preferred_element_type=jnp.float32)
    bias = jnp.concatenate([bg_ref[...]] * B, axis=1)
    out = jnp.maximum(q + w2 + bias, 0.0)

    z = jnp.zeros((n, 1), dtype=out.dtype)
    left = jnp.concatenate([z, out[:, :-1]], axis=1)
    right = jnp.concatenate([out[:, 1:], z], axis=1)
    colt = jax.lax.broadcasted_iota(jnp.int32, (1, BT), 1) % _T
    mfirst = (colt != 0).astype(out.dtype)
    mlast = (colt != _T - 1).astype(out.dtype)
    cw = cw_ref[...]
    y = (cw[:, 1:2] * out
         + cw[:, 0:1] * (mfirst * left)
         + cw[:, 2:3] * (mlast * right)
         + cb_ref[0, 0])
    o_ref[...] = jnp.maximum(y, 0.0)


def kernel(X, A, W, b_gcn, conv_w, conv_b):
    B, N, _, T1 = X.shape
    K, _, T2 = W.shape
    xt = (X.reshape(B, N, T1).transpose(1, 0, 2)
          .reshape(N, B * T1).astype(jnp.bfloat16))
    vmem = pltpu.MemorySpace.VMEM
    y = pl.pallas_call(
        _fused_body,
        in_specs=[
            pl.BlockSpec(memory_space=pl.ANY),
            pl.BlockSpec(memory_space=pl.ANY),
            pl.BlockSpec(memory_space=vmem),
            pl.BlockSpec(memory_space=vmem),
            pl.BlockSpec(memory_space=vmem),
            pl.BlockSpec(memory_space=vmem),
        ],
        out_shape=jax.ShapeDtypeStruct((N, B * T2), jnp.float32),
        scratch_shapes=[
            pltpu.VMEM((N, N), jnp.float32),
            pltpu.VMEM((N, B * T1), jnp.bfloat16),
            pltpu.SemaphoreType.DMA,
            pltpu.SemaphoreType.DMA,
        ],
    )(A, xt, W, b_gcn.reshape(1, T2), conv_w.reshape(1, K),
      conv_b.reshape(1, 1))
    return y.reshape(N, B, T2).transpose(1, 0, 2).reshape(B, N, 1, T2)


# speedup vs baseline: 1.0697x; 1.0697x over previous
import jax
import jax.numpy as jnp
from jax.experimental import pallas as pl

_T = 12
_SEG = 256


def _fused_body(a_ref, x_ref, w_ref, bg_ref, cw_ref, cb_ref, o_ref):
    n = a_ref.shape[0]
    BT = x_ref.shape[1]
    B = BT // _T

    w = w_ref[...]
    rb = jax.lax.broadcasted_iota(jnp.int32, (BT, BT), 0) // _T
    cb = jax.lax.broadcasted_iota(jnp.int32, (BT, BT), 1) // _T
    blockmask = rb == cb
    zseg = jnp.zeros((BT, _SEG - BT), dtype=w.dtype)

    def bd(wk):
        tile = jnp.concatenate([jnp.concatenate([wk] * B, axis=1)] * B, axis=0)
        return jnp.concatenate(
            [jnp.where(blockmask, tile, 0.0), zseg], axis=1)

    WH = jnp.concatenate([bd(w[0] - w[2]), bd(w[1]), bd(w[2])], axis=1)

    x = x_ref[...].astype(jnp.float32)
    ual = jnp.dot(x, WH, preferred_element_type=jnp.float32)
    q = ual[:, 0:BT]
    u1 = ual[:, _SEG:_SEG + BT]
    u2 = ual[:, 2 * _SEG:2 * _SEG + BT]

    A = a_ref[...]
    rown = jax.lax.broadcasted_iota(jnp.int32, (n, n), 0)
    coln = jax.lax.broadcasted_iota(jnp.int32, (n, n), 1)
    diagA = jnp.sum(jnp.where(rown == coln, A, 0.0), axis=1, keepdims=True)
    deg = jnp.sum(A, axis=1, keepdims=True) - diagA
    d = jnp.where(deg > 0, jax.lax.rsqrt(deg), 0.0)

    def smul(v):
        z = d * v
        return -d * (jnp.dot(A, z, preferred_element_type=jnp.float32)
                     - diagA * z)

    w2 = smul(u1 + 2.0 * smul(u2))
    bias = jnp.concatenate([bg_ref[...]] * B, axis=1)
    out = jnp.maximum(q + w2 + bias, 0.0)

    z = jnp.zeros((n, 1), dtype=out.dtype)
    left = jnp.concatenate([z, out[:, :-1]], axis=1)
    right = jnp.concatenate([out[:, 1:], z], axis=1)
    colt = jax.lax.broadcasted_iota(jnp.int32, (1, BT), 1) % _T
    mfirst = (colt != 0).astype(out.dtype)
    mlast = (colt != _T - 1).astype(out.dtype)
    cw = cw_ref[...]
    y = (cw[:, 1:2] * out
         + cw[:, 0:1] * (mfirst * left)
         + cw[:, 2:3] * (mlast * right)
         + cb_ref[0, 0])
    o_ref[...] = jnp.maximum(y, 0.0)


def kernel(X, A, W, b_gcn, conv_w, conv_b):
    B, N, _, T1 = X.shape
    K, _, T2 = W.shape
    xt = (X.reshape(B, N, T1).transpose(1, 0, 2)
          .reshape(N, B * T1).astype(jnp.bfloat16))
    y = pl.pallas_call(
        _fused_body,
        out_shape=jax.ShapeDtypeStruct((N, B * T2), jnp.float32),
    )(A, xt, W, b_gcn.reshape(1, T2), conv_w.reshape(1, K),
      conv_b.reshape(1, 1))
    return y.reshape(N, B, T2).transpose(1, 0, 2).reshape(B, N, 1, T2)


# speedup vs baseline: 1.0772x; 1.0070x over previous
import jax
import jax.numpy as jnp
from jax.experimental import pallas as pl

_T = 12
_SEG = 256


def _fused_body(a_ref, x_ref, w_ref, bg_ref, cw_ref, cb_ref, o_ref):
    n = a_ref.shape[0]
    BT = x_ref.shape[1]
    B = BT // _T

    w = w_ref[...]
    rb = jax.lax.broadcasted_iota(jnp.int32, (BT, BT), 0) // _T
    cb = jax.lax.broadcasted_iota(jnp.int32, (BT, BT), 1) // _T
    blockmask = rb == cb
    zseg = jnp.zeros((BT, _SEG - BT), dtype=w.dtype)

    def bd(wk):
        tile = jnp.concatenate([jnp.concatenate([wk] * B, axis=1)] * B, axis=0)
        return jnp.concatenate(
            [jnp.where(blockmask, tile, 0.0), zseg], axis=1)

    WH = jnp.concatenate([bd(w[0] - w[2]), bd(w[1]), bd(w[2])], axis=1)

    x = x_ref[...]
    ual = jnp.dot(x, WH, preferred_element_type=jnp.float32)
    q = ual[:, 0:BT]
    u1 = ual[:, _SEG:_SEG + BT]
    u2 = ual[:, 2 * _SEG:2 * _SEG + BT]

    A = a_ref[...]
    rown = jax.lax.broadcasted_iota(jnp.int32, (n, n), 0)
    coln = jax.lax.broadcasted_iota(jnp.int32, (n, n), 1)
    A0 = jnp.where(rown == coln, 0.0, A)
    deg = jnp.sum(A0, axis=1, keepdims=True)
    d = jnp.where(deg > 0, jax.lax.rsqrt(deg), 0.0)

    v = -d * jnp.dot(A0, d * u2, preferred_element_type=jnp.float32)
    p = u1 + 2.0 * v
    w2 = -d * jnp.dot(A0, d * p, preferred_element_type=jnp.float32)
    bias = jnp.concatenate([bg_ref[...]] * B, axis=1)
    out = jnp.maximum(q + w2 + bias, 0.0)

    z = jnp.zeros((n, 1), dtype=out.dtype)
    left = jnp.concatenate([z, out[:, :-1]], axis=1)
    right = jnp.concatenate([out[:, 1:], z], axis=1)
    colt = jax.lax.broadcasted_iota(jnp.int32, (1, BT), 1) % _T
    mfirst = (colt != 0).astype(out.dtype)
    mlast = (colt != _T - 1).astype(out.dtype)
    cw = cw_ref[...]
    y = (cw[:, 1:2] * out
         + cw[:, 0:1] * (mfirst * left)
         + cw[:, 2:3] * (mlast * right)
         + cb_ref[0, 0])
    o_ref[...] = jnp.maximum(y, 0.0)


def kernel(X, A, W, b_gcn, conv_w, conv_b):
    B, N, _, T1 = X.shape
    K, _, T2 = W.shape
    xt = X.reshape(B, N, T1).transpose(1, 0, 2).reshape(N, B * T1)
    y = pl.pallas_call(
        _fused_body,
        out_shape=jax.ShapeDtypeStruct((N, B * T2), X.dtype),
    )(A, xt, W, b_gcn.reshape(1, T2), conv_w.reshape(1, K),
      conv_b.reshape(1, 1))
    return y.reshape(N, B, T2).transpose(1, 0, 2).reshape(B, N, 1, T2)
